# baseline (speedup 1.0000x reference)
import functools

import jax
import jax.numpy as jnp
import numpy as np
from jax import lax
from jax.experimental import pallas as pl
from jax.experimental.pallas import tpu as pltpu
from jax.experimental.pallas import tpu_sc as plsc

_SRC = 600.0
_LOGIT = float(np.log(0.7) - np.log(0.3))

_A, _H, _W = 9, 48, 48
_L = 16
_JB = 3
_ROWS = _H // _JB
_UE = _ROWS * _W
_NU = _A * _JB
_PL = _H * _W


def kernel(cla_map, reg_map, anchor):
    mesh = plsc.VectorSubcoreMesh(core_axis_name="c", subcore_axis_name="s",
                                  num_cores=2, num_subcores=16)

    @functools.partial(
        pl.kernel,
        out_type=jax.ShapeDtypeStruct((_A * _H * _W, 8), jnp.float32),
        mesh=mesh,
        compiler_params=pltpu.CompilerParams(needs_layout_passes=False),
        scratch_types=[
            pltpu.VMEM((_PL + _UE,), jnp.float32),
            pltpu.VMEM((3 * _PL + _UE,), jnp.float32),
            pltpu.VMEM((4 * 377,), jnp.float32),
            pltpu.VMEM((_UE, 8), jnp.float32),
            pltpu.SemaphoreType.DMA,
            pltpu.SemaphoreType.DMA,
        ],
    )
    def sc_kernel(cla, reg, anc, out,
                  cbuf, rbuf, aux, obuf, isem, osem):
        wid = lax.axis_index("s") * 2 + lax.axis_index("c")

        @pl.when(wid < _NU)
        def _():
            p = wid // _JB
            j0 = (wid % _JB) * _ROWS
            iota = lax.iota(jnp.int32, _L)

            cbase = (2 * p) * _PL + j0 * _W
            rbase = (4 * p) * _PL + j0 * _W
            hs = [
                pltpu.async_copy(cla.at[pl.ds(cbase, _PL + _UE)], cbuf, isem),
                pltpu.async_copy(reg.at[pl.ds(rbase, 3 * _PL + _UE)], rbuf, isem),
                pltpu.async_copy(anc, aux, isem),
            ]
            for h in hs:
                h.wait()

            avec = []
            for g in range(3):
                kv = (iota + (g * _L)) * (4 * p)
                avec.append(tuple(
                    plsc.load_gather(aux, [kv + c]) for c in range(4)))

            ohs = []
            for j in range(_ROWS):
                for g in range(3):
                    acx, acy, aw, ah = avec[g]
                    off = j * _W + g * _L
                    c0v = cbuf[pl.ds(off, _L)]
                    c1v = cbuf[pl.ds(_PL + off, _L)]
                    cx = (rbuf[pl.ds(off, _L)] * aw + acx) * _SRC
                    cy = (rbuf[pl.ds(_PL + off, _L)] * ah + acy) * _SRC
                    wv = jnp.exp(rbuf[pl.ds(2 * _PL + off, _L)]) * aw * _SRC
                    hv = jnp.exp(rbuf[pl.ds(3 * _PL + off, _L)]) * ah * _SRC
                    wh = wv * 0.5
                    hh = hv * 0.5
                    ltx = cx - wh
                    lty = cy - hh
                    rbx = cx + wh
                    rby = cy + hh
                    m = ((c1v - c0v > _LOGIT)
                         & (ltx >= 0.0) & (lty >= 0.0)
                         & (rbx <= _SRC) & (rby <= _SRC))
                    rows = iota + (j * _W + g * _L)
                    vals = (ltx, lty, rbx, rby,
                            cx * (1.0 / _SRC), cy * (1.0 / _SRC),
                            wv * (1.0 / _SRC), hv * (1.0 / _SRC))
                    for c, v in enumerate(vals):
                        plsc.store_scatter(
                            obuf, [rows, jnp.full((_L,), c, jnp.int32)],
                            jnp.where(m, v, 0.0))
                if j % 4 == 3:
                    ohs.append(pltpu.async_copy(
                        obuf.at[pl.ds((j - 3) * _W, 4 * _W)],
                        out.at[pl.ds((p * _H + j0 + j - 3) * _W, 4 * _W)],
                        osem))
            for h in ohs:
                h.wait()

    anc = lax.slice(anchor, (0, 2), (49 * 376 + 1, 6), (49, 1)).reshape(-1)
    return sc_kernel(cla_map.reshape(-1), reg_map.reshape(-1), anc)


# speedup vs baseline: 1.0739x; 1.0739x over previous
import functools

import jax
import jax.numpy as jnp
import numpy as np
from jax import lax
from jax.experimental import pallas as pl
from jax.experimental.pallas import tpu as pltpu
from jax.experimental.pallas import tpu_sc as plsc

_SRC = 600.0
_LOGIT = float(np.log(0.7) - np.log(0.3))

_A, _H, _W = 9, 48, 48
_L = 16
_JB = 3
_ROWS = _H // _JB
_UE = _ROWS * _W
_NU = _A * _JB
_PL = _H * _W


def kernel(cla_map, reg_map, anchor):
    mesh = plsc.VectorSubcoreMesh(core_axis_name="c", subcore_axis_name="s",
                                  num_cores=2, num_subcores=16)

    @functools.partial(
        pl.kernel,
        out_type=jax.ShapeDtypeStruct((_A * _H * _W, 8), jnp.float32),
        mesh=mesh,
        compiler_params=pltpu.CompilerParams(needs_layout_passes=False),
        scratch_types=[
            pltpu.VMEM((2, _ROWS, _W), jnp.float32),
            pltpu.VMEM((4, _ROWS, _W), jnp.float32),
            pltpu.VMEM((4 * 377,), jnp.float32),
            pltpu.VMEM((_UE, 8), jnp.float32),
            pltpu.SemaphoreType.DMA,
            pltpu.SemaphoreType.DMA,
        ],
    )
    def sc_kernel(cla, reg, anc, out,
                  c01, treg, aux, obuf, isem, osem):
        wid = lax.axis_index("s") * 2 + lax.axis_index("c")

        @pl.when(wid < _NU)
        def _():
            p = wid // _JB
            j0 = (wid % _JB) * _ROWS
            iota = lax.iota(jnp.int32, _L)

            ha = pltpu.async_copy(anc, aux, osem)
            hs = [
                pltpu.async_copy(cla.at[0, pl.ds(2 * p, 2), pl.ds(j0, _ROWS)],
                                 c01, isem),
                pltpu.async_copy(reg.at[0, pl.ds(4 * p, 4), pl.ds(j0, _ROWS)],
                                 treg, isem),
            ]
            ha.wait()
            avec = []
            for g in range(3):
                kv = (iota + (g * _L)) * (4 * p)
                avec.append(tuple(
                    plsc.load_gather(aux, [kv + c]) for c in range(4)))
            for h in hs:
                h.wait()

            ohs = []
            for j in range(_ROWS):
                for g in range(3):
                    acx, acy, aw, ah = avec[g]
                    sl = pl.ds(g * _L, _L)
                    c0v = c01[0, j, sl]
                    c1v = c01[1, j, sl]
                    cx = (treg[0, j, sl] * aw + acx) * _SRC
                    cy = (treg[1, j, sl] * ah + acy) * _SRC
                    wv = jnp.exp(treg[2, j, sl]) * aw * _SRC
                    hv = jnp.exp(treg[3, j, sl]) * ah * _SRC
                    wh = wv * 0.5
                    hh = hv * 0.5
                    ltx = cx - wh
                    lty = cy - hh
                    rbx = cx + wh
                    rby = cy + hh
                    m = ((c1v - c0v > _LOGIT)
                         & (ltx >= 0.0) & (lty >= 0.0)
                         & (rbx <= _SRC) & (rby <= _SRC))
                    rows = iota + (j * _W + g * _L)
                    vals = (ltx, lty, rbx, rby,
                            cx * (1.0 / _SRC), cy * (1.0 / _SRC),
                            wv * (1.0 / _SRC), hv * (1.0 / _SRC))
                    for c, v in enumerate(vals):
                        plsc.store_scatter(
                            obuf, [rows, jnp.full((_L,), c, jnp.int32)],
                            jnp.where(m, v, 0.0))
                if j % 4 == 3:
                    ohs.append(pltpu.async_copy(
                        obuf.at[pl.ds((j - 3) * _W, 4 * _W)],
                        out.at[pl.ds((p * _H + j0 + j - 3) * _W, 4 * _W)],
                        osem))
            for h in ohs:
                h.wait()

    anc = lax.slice(anchor, (0, 2), (49 * 376 + 1, 6), (49, 1)).reshape(-1)
    return sc_kernel(cla_map, reg_map, anc)
